# baseline (speedup 1.0000x reference)
import functools

import jax
import jax.numpy as jnp
from jax import lax
from jax.experimental import pallas as pl
from jax.experimental.pallas import tpu as pltpu
from jax.experimental.pallas import tpu_sc as plsc

B = 16384
D = 64
N_CORES = 2
N_SUBCORES = 16
N_WORKERS = N_CORES * N_SUBCORES
BPW = B // N_WORKERS
CHUNK = 128
NCHUNK = BPW // CHUNK
LANES = 16
NBLK = BPW // LANES


def _tec_body(h_hbm, t_hbm, r_hbm, ent_hbm, rel_hbm, out_hbm,
              hidx_v, tidx_v, ridx_v, he_v, te_v, re_v, out_v, sem):
    cid = lax.axis_index("c")
    sid = lax.axis_index("s")
    wid = sid * N_CORES + cid
    base = wid * BPW

    pltpu.sync_copy(h_hbm.at[pl.ds(base, BPW)], hidx_v)
    pltpu.sync_copy(t_hbm.at[pl.ds(base, BPW)], tidx_v)
    pltpu.sync_copy(r_hbm.at[pl.ds(base, BPW)], ridx_v)

    copies = []
    for j in range(NCHUNK):
        sl = pl.ds(j * CHUNK, CHUNK)
        copies.append(pltpu.async_copy(
            ent_hbm.at[hidx_v.at[sl]], he_v.at[sl], sem))
        copies.append(pltpu.async_copy(
            ent_hbm.at[tidx_v.at[sl]], te_v.at[sl], sem))
        copies.append(pltpu.async_copy(
            rel_hbm.at[ridx_v.at[sl]], re_v.at[sl], sem))
    for c in copies:
        c.wait()

    lane = jnp.arange(LANES, dtype=jnp.int32)

    def block(b, carry):
        rows = b * LANES + lane

        def dcol(d, acc):
            col = jnp.full((LANES,), d, dtype=jnp.int32)
            hv = plsc.load_gather(he_v, [rows, col])
            tv = plsc.load_gather(te_v, [rows, col])
            rv = plsc.load_gather(re_v, [rows, col])
            e = hv + rv - tv
            return acc + e * e

        s = lax.fori_loop(0, D, dcol, jnp.zeros((LANES,), jnp.float32))
        s = s + jnp.float32(1e-12)
        i = plsc.bitcast(s, jnp.int32)
        y = plsc.bitcast(jnp.int32(0x5F3759DF) - (i >> 1), jnp.float32)
        half_s = jnp.float32(0.5) * s
        for _ in range(3):
            y = y * (jnp.float32(1.5) - half_s * y * y)
        out_v[pl.ds(b * LANES, LANES)] = -(s * y)
        return carry

    lax.fori_loop(0, NBLK, block, 0)
    pltpu.sync_copy(out_v, out_hbm.at[pl.ds(base, BPW)])


@jax.jit
def _score(triples, entity_emb, relation_emb):
    mesh = plsc.VectorSubcoreMesh(core_axis_name="c", subcore_axis_name="s")
    run = functools.partial(
        pl.kernel,
        mesh=mesh,
        compiler_params=pltpu.CompilerParams(
            needs_layout_passes=False, use_tc_tiling_on_sc=False),
        out_type=jax.ShapeDtypeStruct((B,), jnp.float32),
        scratch_types=[
            pltpu.VMEM((BPW,), jnp.int32),
            pltpu.VMEM((BPW,), jnp.int32),
            pltpu.VMEM((BPW,), jnp.int32),
            pltpu.VMEM((BPW, D), jnp.float32),
            pltpu.VMEM((BPW, D), jnp.float32),
            pltpu.VMEM((BPW, D), jnp.float32),
            pltpu.VMEM((BPW,), jnp.float32),
            pltpu.SemaphoreType.DMA,
        ],
    )(_tec_body)
    return run(triples[0], triples[1], triples[2], entity_emb, relation_emb)


def kernel(triples, entity_emb, relation_emb):
    return _score(triples.astype(jnp.int32), entity_emb, relation_emb)


# speedup vs baseline: 1.5260x; 1.5260x over previous
import functools

import jax
import jax.numpy as jnp
from jax import lax
from jax.experimental import pallas as pl
from jax.experimental.pallas import tpu as pltpu
from jax.experimental.pallas import tpu_sc as plsc

B = 16384
D = 64
N_CORES = 2
N_SUBCORES = 16
N_WORKERS = N_CORES * N_SUBCORES
BPW = B // N_WORKERS
LANES = 16
CH = 256
NPASS = BPW // CH
NBLK = CH // LANES


def _tec_body(h_hbm, t_hbm, r_hbm, ent_hbm, rel_hbm, dummy_hbm, out_hbm,
              hidx_v, tidx_v, ridx_v, he_v, te_v, re_v, out_v, sem):
    cid = lax.axis_index("c")
    sid = lax.axis_index("s")
    wid = sid * N_CORES + cid
    base = wid * BPW

    pltpu.sync_copy(h_hbm.at[pl.ds(base, BPW)], hidx_v)
    pltpu.sync_copy(t_hbm.at[pl.ds(base, BPW)], tidx_v)
    pltpu.sync_copy(r_hbm.at[pl.ds(base, BPW)], ridx_v)

    lane = jnp.arange(LANES, dtype=jnp.int32)

    def half(p, carry0):
        def fire(g, carry):
            off = p * CH + g * LANES
            hv = hidx_v[pl.ds(off, LANES)]
            tv = tidx_v[pl.ds(off, LANES)]
            rv = ridx_v[pl.ds(off, LANES)]
            for k in range(LANES):
                dst = pl.ds(g * LANES + k, 1)
                pltpu.async_copy(ent_hbm.at[pl.ds(hv[k], 1), :],
                                 he_v.at[dst, :], sem)
                pltpu.async_copy(ent_hbm.at[pl.ds(tv[k], 1), :],
                                 te_v.at[dst, :], sem)
                pltpu.async_copy(rel_hbm.at[pl.ds(rv[k], 1), :],
                                 re_v.at[dst, :], sem)
            return carry

        lax.fori_loop(0, CH // LANES, fire, 0)
        pltpu.make_async_copy(dummy_hbm, he_v, sem).wait()
        pltpu.make_async_copy(dummy_hbm, te_v, sem).wait()
        pltpu.make_async_copy(dummy_hbm, re_v, sem).wait()

        def block(b, carry):
            rows = b * LANES + lane

            def dcol(d, acc):
                col = jnp.full((LANES,), d, dtype=jnp.int32)
                hv = plsc.load_gather(he_v, [rows, col])
                tv = plsc.load_gather(te_v, [rows, col])
                rv = plsc.load_gather(re_v, [rows, col])
                e = hv + rv - tv
                return acc + e * e

            s = lax.fori_loop(0, D, dcol, jnp.zeros((LANES,), jnp.float32))
            s = s + jnp.float32(1e-12)
            i = plsc.bitcast(s, jnp.int32)
            y = plsc.bitcast(jnp.int32(0x5F3759DF) - (i >> 1), jnp.float32)
            half_s = jnp.float32(0.5) * s
            for _ in range(3):
                y = y * (jnp.float32(1.5) - half_s * y * y)
            out_v[pl.ds(p * CH + b * LANES, LANES)] = -(s * y)
            return carry

        lax.fori_loop(0, NBLK, block, 0)
        return carry0

    lax.fori_loop(0, NPASS, half, 0)
    pltpu.sync_copy(out_v, out_hbm.at[pl.ds(base, BPW)])


@jax.jit
def _score(triples, entity_emb, relation_emb):
    mesh = plsc.VectorSubcoreMesh(core_axis_name="c", subcore_axis_name="s")
    run = functools.partial(
        pl.kernel,
        mesh=mesh,
        compiler_params=pltpu.CompilerParams(needs_layout_passes=False),
        out_type=jax.ShapeDtypeStruct((B,), jnp.float32),
        scratch_types=[
            pltpu.VMEM((BPW,), jnp.int32),
            pltpu.VMEM((BPW,), jnp.int32),
            pltpu.VMEM((BPW,), jnp.int32),
            pltpu.VMEM((CH, D), jnp.float32),
            pltpu.VMEM((CH, D), jnp.float32),
            pltpu.VMEM((CH, D), jnp.float32),
            pltpu.VMEM((BPW,), jnp.float32),
            pltpu.SemaphoreType.DMA,
        ],
    )(_tec_body)
    dummy = jnp.zeros((CH, D), jnp.float32)
    return run(triples[0], triples[1], triples[2], entity_emb, relation_emb,
               dummy)


def kernel(triples, entity_emb, relation_emb):
    return _score(triples.astype(jnp.int32), entity_emb, relation_emb)
